# Initial kernel scaffold; baseline (speedup 1.0000x reference)
import functools

import jax
import jax.numpy as jnp
from jax import lax
from jax.experimental import pallas as pl
from jax.experimental.pallas import tpu as pltpu
from jax.experimental.pallas import tpu_sc as plsc

N = 10000
E = 320000
D_IN = 128
D_EDGE = 16
D_OUT = 128

NC = 2
NS = 16
NW = NC * NS
E_PER_W = E // NW
CHUNK = 80
STEPS = E_PER_W // CHUNK
NROWS2D = E // CHUNK
ROWS_PER_SUB = N // NS

BLK = 1000
GRID = N // BLK


def _mm_body(x_ref, w_ref, o_ref):
    o_ref[...] = jnp.dot(x_ref[...], w_ref[...],
                         preferred_element_type=jnp.float32)


def _tc_matmul(x, wT):
    return pl.pallas_call(
        _mm_body,
        grid=(GRID,),
        in_specs=[
            pl.BlockSpec((BLK, D_IN), lambda i: (i, 0)),
            pl.BlockSpec((D_IN, D_OUT), lambda i: (0, 0)),
        ],
        out_specs=pl.BlockSpec((BLK, D_OUT), lambda i: (i, 0)),
        out_shape=jax.ShapeDtypeStruct((N, D_OUT), jnp.float32),
    )(x, wT)


def _combine_body(p0, p1, s0, s1, x_ref, wself, we, o_ref):
    s = s0[...] + s1[...]
    o_ref[...] = (
        p0[...] + p1[...]
        + jnp.dot(x_ref[...], wself[...], preferred_element_type=jnp.float32)
        + jnp.dot(s, we[...], preferred_element_type=jnp.float32)
    )


def _tc_combine(p0, p1, s0, s1, x, wselfT, weT):
    return pl.pallas_call(
        _combine_body,
        grid=(GRID,),
        in_specs=[
            pl.BlockSpec((BLK, D_OUT), lambda i: (i, 0)),
            pl.BlockSpec((BLK, D_OUT), lambda i: (i, 0)),
            pl.BlockSpec((BLK, D_EDGE), lambda i: (i, 0)),
            pl.BlockSpec((BLK, D_EDGE), lambda i: (i, 0)),
            pl.BlockSpec((BLK, D_IN), lambda i: (i, 0)),
            pl.BlockSpec((D_IN, D_OUT), lambda i: (0, 0)),
            pl.BlockSpec((D_EDGE, D_OUT), lambda i: (0, 0)),
        ],
        out_specs=pl.BlockSpec((BLK, D_OUT), lambda i: (i, 0)),
        out_shape=jax.ShapeDtypeStruct((N, D_OUT), jnp.float32),
    )(p0, p1, s0, s1, x, wselfT, weT)


def _sc_body(y, src2d, dst2d, ea3d, zp, zs, p_out, s_out,
             src_v, dst_v, rows_v, ea_v, acc, acc_s, sem):
    cid = lax.axis_index("c")
    sid = lax.axis_index("s")
    wid = sid * NC + cid

    r0 = sid * ROWS_PER_SUB
    pltpu.sync_copy(zp.at[pl.ds(r0, ROWS_PER_SUB)],
                    acc.at[pl.ds(r0, ROWS_PER_SUB)])
    pltpu.sync_copy(zs.at[pl.ds(r0, ROWS_PER_SUB)],
                    acc_s.at[pl.ds(r0, ROWS_PER_SUB)])
    plsc.subcore_barrier()

    c0 = wid * STEPS
    pltpu.sync_copy(src2d.at[pl.ds(c0, STEPS)], src_v)
    pltpu.sync_copy(dst2d.at[pl.ds(c0, STEPS)], dst_v)

    def body(t, carry):
        pltpu.async_copy(y.at[src_v.at[t]], rows_v, sem).wait()
        pltpu.sync_copy(ea3d.at[c0 + t], ea_v)
        pltpu.sync_copy(rows_v, acc.at[dst_v.at[t]], add=True)
        pltpu.sync_copy(ea_v, acc_s.at[dst_v.at[t]], add=True)
        return carry

    lax.fori_loop(0, STEPS, body, 0)
    plsc.subcore_barrier()

    pltpu.sync_copy(acc.at[pl.ds(r0, ROWS_PER_SUB)],
                    p_out.at[cid, pl.ds(r0, ROWS_PER_SUB)])
    pltpu.sync_copy(acc_s.at[pl.ds(r0, ROWS_PER_SUB)],
                    s_out.at[cid, pl.ds(r0, ROWS_PER_SUB)])


_sc_scatter = functools.partial(
    pl.kernel,
    out_type=[
        jax.ShapeDtypeStruct((NC, N, D_OUT), jnp.float32),
        jax.ShapeDtypeStruct((NC, N, D_EDGE), jnp.float32),
    ],
    mesh=plsc.VectorSubcoreMesh(core_axis_name="c", subcore_axis_name="s"),
    scratch_types=[
        pltpu.VMEM((STEPS, CHUNK), jnp.int32),
        pltpu.VMEM((STEPS, CHUNK), jnp.int32),
        pltpu.VMEM((CHUNK, D_OUT), jnp.float32),
        pltpu.VMEM((CHUNK, D_EDGE), jnp.float32),
        pltpu.VMEM_SHARED((N, D_OUT), jnp.float32),
        pltpu.VMEM_SHARED((N, D_EDGE), jnp.float32),
        pltpu.SemaphoreType.DMA,
    ],
)(_sc_body)


def kernel(x, edge_index, edge_attr, W_msg, W_self):
    wxT = W_msg[:, :D_IN].T
    weT = W_msg[:, D_IN:].T
    wselfT = W_self.T
    src2d = edge_index[0].reshape(NROWS2D, CHUNK)
    dst2d = edge_index[1].reshape(NROWS2D, CHUNK)
    ea3d = edge_attr.reshape(NROWS2D, CHUNK, D_EDGE)
    zp = jnp.zeros((N, D_OUT), jnp.float32)
    zs = jnp.zeros((N, D_EDGE), jnp.float32)

    y = _tc_matmul(x, wxT)
    p, s = _sc_scatter(y, src2d, dst2d, ea3d, zp, zs)
    return _tc_combine(p[0], p[1], s[0], s[1], x, wselfT, weT)


# speedup vs baseline: 3.1924x; 3.1924x over previous
import functools

import jax
import jax.numpy as jnp
from jax import lax
from jax.experimental import pallas as pl
from jax.experimental.pallas import tpu as pltpu
from jax.experimental.pallas import tpu_sc as plsc

N = 10000
E = 320000
D_IN = 128
D_EDGE = 16
D_OUT = 128
D_HALF = D_OUT // 2

NC = 2
NS = 16
E_PER_SUB = E // NS
CHUNK = 80
STEPS = E_PER_SUB // CHUNK
NROWS2D = E // CHUNK
NPAD = 10240
ROWS_PER_SUB = NPAD // NS

BLK = 1000
GRID = N // BLK


def _mm_body(x_ref, w0_ref, w1_ref, o0_ref, o1_ref):
    xv = x_ref[...]
    o0_ref[...] = jnp.dot(xv, w0_ref[...], preferred_element_type=jnp.float32)
    o1_ref[...] = jnp.dot(xv, w1_ref[...], preferred_element_type=jnp.float32)


def _tc_matmul_halves(x, wT):
    return pl.pallas_call(
        _mm_body,
        grid=(GRID,),
        in_specs=[
            pl.BlockSpec((BLK, D_IN), lambda i: (i, 0)),
            pl.BlockSpec((D_IN, D_HALF), lambda i: (0, 0)),
            pl.BlockSpec((D_IN, D_HALF), lambda i: (0, 0)),
        ],
        out_specs=[
            pl.BlockSpec((BLK, D_HALF), lambda i: (i, 0)),
            pl.BlockSpec((BLK, D_HALF), lambda i: (i, 0)),
        ],
        out_shape=[
            jax.ShapeDtypeStruct((N, D_HALF), jnp.float32),
            jax.ShapeDtypeStruct((N, D_HALF), jnp.float32),
        ],
    )(x, wT[:, :D_HALF], wT[:, D_HALF:])


def _combine_body(p0, p1, s0, x_ref, wself, we, o_ref):
    agg = jnp.concatenate([p0[...], p1[...]], axis=-1)
    o_ref[...] = (
        agg
        + jnp.dot(x_ref[...], wself[...], preferred_element_type=jnp.float32)
        + jnp.dot(s0[...], we[...], preferred_element_type=jnp.float32)
    )


def _tc_combine(p0, p1, s0, x, wselfT, weT):
    return pl.pallas_call(
        _combine_body,
        grid=(GRID,),
        in_specs=[
            pl.BlockSpec((BLK, D_HALF), lambda i: (i, 0)),
            pl.BlockSpec((BLK, D_HALF), lambda i: (i, 0)),
            pl.BlockSpec((BLK, D_EDGE), lambda i: (i, 0)),
            pl.BlockSpec((BLK, D_IN), lambda i: (i, 0)),
            pl.BlockSpec((D_IN, D_OUT), lambda i: (0, 0)),
            pl.BlockSpec((D_EDGE, D_OUT), lambda i: (0, 0)),
        ],
        out_specs=pl.BlockSpec((BLK, D_OUT), lambda i: (i, 0)),
        out_shape=jax.ShapeDtypeStruct((N, D_OUT), jnp.float32),
    )(p0, p1, s0, x, wselfT, weT)


def _sc_body(y0, y1, src3d, dst3d, ea3d, zp, zs, p_out, s_out,
             src_v, dst_v, rows_v, ea_v, acc, acc_s, sem):
    cid = lax.axis_index("c")
    sid = lax.axis_index("s")

    r0 = sid * ROWS_PER_SUB
    pltpu.sync_copy(zp.at[pl.ds(r0, ROWS_PER_SUB)],
                    acc.at[pl.ds(r0, ROWS_PER_SUB)])

    @pl.when(cid == 0)
    def _():
        pltpu.sync_copy(zs.at[pl.ds(r0, ROWS_PER_SUB)],
                        acc_s.at[pl.ds(r0, ROWS_PER_SUB)])

    plsc.subcore_barrier()

    c0 = sid * STEPS
    pltpu.sync_copy(src3d.at[sid], src_v)
    pltpu.sync_copy(dst3d.at[sid], dst_v)

    def run_half(y_ref, with_ea):
        def body(t, carry):
            pltpu.async_copy(y_ref.at[src_v.at[t]], rows_v, sem).wait()
            pltpu.sync_copy(rows_v, acc.at[dst_v.at[t]], add=True)
            if with_ea:
                pltpu.sync_copy(ea3d.at[c0 + t], ea_v)
                pltpu.sync_copy(ea_v, acc_s.at[dst_v.at[t]], add=True)
            return carry
        lax.fori_loop(0, STEPS, body, 0)

    @pl.when(cid == 0)
    def _():
        run_half(y0, True)

    @pl.when(cid == 1)
    def _():
        run_half(y1, False)

    plsc.subcore_barrier()

    pltpu.sync_copy(acc.at[pl.ds(r0, ROWS_PER_SUB)],
                    p_out.at[cid, pl.ds(r0, ROWS_PER_SUB)])

    @pl.when(cid == 0)
    def _():
        pltpu.sync_copy(acc_s.at[pl.ds(r0, ROWS_PER_SUB)],
                        s_out.at[pl.ds(r0, ROWS_PER_SUB)])


_sc_scatter = functools.partial(
    pl.kernel,
    out_type=[
        jax.ShapeDtypeStruct((NC, NPAD, D_HALF), jnp.float32),
        jax.ShapeDtypeStruct((NPAD, D_EDGE), jnp.float32),
    ],
    mesh=plsc.VectorSubcoreMesh(core_axis_name="c", subcore_axis_name="s"),
    compiler_params=pltpu.CompilerParams(use_tc_tiling_on_sc=False),
    scratch_types=[
        pltpu.VMEM((STEPS, CHUNK), jnp.int32),
        pltpu.VMEM((STEPS, CHUNK), jnp.int32),
        pltpu.VMEM((CHUNK, D_HALF), jnp.float32),
        pltpu.VMEM((CHUNK, D_EDGE), jnp.float32),
        pltpu.VMEM_SHARED((NPAD, D_HALF), jnp.float32),
        pltpu.VMEM_SHARED((NPAD, D_EDGE), jnp.float32),
        pltpu.SemaphoreType.DMA,
    ],
)(_sc_body)


def kernel(x, edge_index, edge_attr, W_msg, W_self):
    wxT = W_msg[:, :D_IN].T
    weT = W_msg[:, D_IN:].T
    wselfT = W_self.T
    src3d = edge_index[0].reshape(NS, STEPS, CHUNK)
    dst3d = edge_index[1].reshape(NS, STEPS, CHUNK)
    ea3d = edge_attr.reshape(NROWS2D, CHUNK, D_EDGE)
    zp = jnp.zeros((NPAD, D_HALF), jnp.float32)
    zs = jnp.zeros((NPAD, D_EDGE), jnp.float32)

    y0, y1 = _tc_matmul_halves(x, wxT)
    p, s = _sc_scatter(y0, y1, src3d, dst3d, ea3d, zp, zs)
    return _tc_combine(p[0, :N], p[1, :N], s[:N], x, wselfT, weT)


# speedup vs baseline: 5.4393x; 1.7038x over previous
import functools

import jax
import jax.numpy as jnp
from jax import lax
from jax.experimental import pallas as pl
from jax.experimental.pallas import tpu as pltpu
from jax.experimental.pallas import tpu_sc as plsc

N = 10000
E = 320000
D_IN = 128
D_EDGE = 16
D_OUT = 128
D_HALF = D_OUT // 2

NC = 2
NS = 16
E_PER_SUB = E // NS
CHUNK = 80
STEPS = E_PER_SUB // CHUNK
K = 5
NROWS2D = E // CHUNK
NPAD = 10240
ROWS_PER_SUB = NPAD // NS

BLK = 1000
GRID = N // BLK


def _mm_body(x_ref, w0_ref, w1_ref, o0_ref, o1_ref):
    xv = x_ref[...]
    o0_ref[...] = jnp.dot(xv, w0_ref[...], preferred_element_type=jnp.float32)
    o1_ref[...] = jnp.dot(xv, w1_ref[...], preferred_element_type=jnp.float32)


def _tc_matmul_halves(x, wT):
    return pl.pallas_call(
        _mm_body,
        grid=(GRID,),
        in_specs=[
            pl.BlockSpec((BLK, D_IN), lambda i: (i, 0)),
            pl.BlockSpec((D_IN, D_HALF), lambda i: (0, 0)),
            pl.BlockSpec((D_IN, D_HALF), lambda i: (0, 0)),
        ],
        out_specs=[
            pl.BlockSpec((BLK, D_HALF), lambda i: (i, 0)),
            pl.BlockSpec((BLK, D_HALF), lambda i: (i, 0)),
        ],
        out_shape=[
            jax.ShapeDtypeStruct((N, D_HALF), jnp.float32),
            jax.ShapeDtypeStruct((N, D_HALF), jnp.float32),
        ],
    )(x, wT[:, :D_HALF], wT[:, D_HALF:])


def _combine_body(p0, p1, s0, s1, x_ref, wself, we, o_ref):
    agg = jnp.concatenate([p0[...], p1[...]], axis=-1)
    o_ref[...] = (
        agg
        + jnp.dot(x_ref[...], wself[...], preferred_element_type=jnp.float32)
        + jnp.dot(s0[...] + s1[...], we[...],
                  preferred_element_type=jnp.float32)
    )


def _tc_combine(p0, p1, s0, s1, x, wselfT, weT):
    return pl.pallas_call(
        _combine_body,
        grid=(GRID,),
        in_specs=[
            pl.BlockSpec((BLK, D_HALF), lambda i: (i, 0)),
            pl.BlockSpec((BLK, D_HALF), lambda i: (i, 0)),
            pl.BlockSpec((BLK, D_EDGE), lambda i: (i, 0)),
            pl.BlockSpec((BLK, D_EDGE), lambda i: (i, 0)),
            pl.BlockSpec((BLK, D_IN), lambda i: (i, 0)),
            pl.BlockSpec((D_IN, D_OUT), lambda i: (0, 0)),
            pl.BlockSpec((D_EDGE, D_OUT), lambda i: (0, 0)),
        ],
        out_specs=pl.BlockSpec((BLK, D_OUT), lambda i: (i, 0)),
        out_shape=jax.ShapeDtypeStruct((N, D_OUT), jnp.float32),
    )(p0, p1, s0, s1, x, wselfT, weT)


def _sc_body(y0, y1, src3d, dst3d, ea3d, zp, zs, p_out, s_out,
             src_v, dst_v, rows0_v, rows1_v, rows2_v, rows3_v, rows4_v,
             ea_v, acc, acc_s, sem0, sem1, sem2, sem3, sem4):
    cid = lax.axis_index("c")
    sid = lax.axis_index("s")

    r0 = sid * ROWS_PER_SUB
    pltpu.sync_copy(zp.at[pl.ds(r0, ROWS_PER_SUB)],
                    acc.at[pl.ds(r0, ROWS_PER_SUB)])
    pltpu.sync_copy(zs.at[pl.ds(r0, ROWS_PER_SUB)],
                    acc_s.at[pl.ds(r0, ROWS_PER_SUB)])
    plsc.subcore_barrier()

    c0 = sid * STEPS
    pltpu.sync_copy(src3d.at[sid], src_v)
    pltpu.sync_copy(dst3d.at[sid], dst_v)

    def run_half(y_ref, parity):
        rows = [rows0_v, rows1_v, rows2_v, rows3_v, rows4_v]
        sems = [sem0, sem1, sem2, sem3, sem4]

        def fire(t, b):
            pltpu.async_copy(y_ref.at[src_v.at[t]], rows[b], sems[b])

        def step(t, b, refill):
            pltpu.make_async_copy(y_ref.at[src_v.at[0]], rows[b],
                                  sems[b]).wait()
            pltpu.sync_copy(rows[b], acc.at[dst_v.at[t]], add=True)
            if refill:
                fire(t + K, b)

            @pl.when(t % 2 == parity)
            def _():
                pltpu.sync_copy(ea3d.at[c0 + t], ea_v)
                pltpu.sync_copy(ea_v, acc_s.at[dst_v.at[t]], add=True)

        for b in range(K):
            fire(b, b)

        def body(i, carry):
            t0 = i * K
            for b in range(K):
                step(t0 + b, b, True)
            return carry

        lax.fori_loop(0, STEPS // K - 1, body, 0)
        for b in range(K):
            step(STEPS - K + b, b, False)

    @pl.when(cid == 0)
    def _():
        run_half(y0, 0)

    @pl.when(cid == 1)
    def _():
        run_half(y1, 1)

    plsc.subcore_barrier()

    pltpu.sync_copy(acc.at[pl.ds(r0, ROWS_PER_SUB)],
                    p_out.at[cid, pl.ds(r0, ROWS_PER_SUB)])

    pltpu.sync_copy(acc_s.at[pl.ds(r0, ROWS_PER_SUB)],
                    s_out.at[cid, pl.ds(r0, ROWS_PER_SUB)])


_sc_scatter = functools.partial(
    pl.kernel,
    out_type=[
        jax.ShapeDtypeStruct((NC, NPAD, D_HALF), jnp.float32),
        jax.ShapeDtypeStruct((NC, NPAD, D_EDGE), jnp.float32),
    ],
    mesh=plsc.VectorSubcoreMesh(core_axis_name="c", subcore_axis_name="s"),
    compiler_params=pltpu.CompilerParams(use_tc_tiling_on_sc=False),
    scratch_types=[
        pltpu.VMEM((STEPS, CHUNK), jnp.int32),
        pltpu.VMEM((STEPS, CHUNK), jnp.int32),
        pltpu.VMEM((CHUNK, D_HALF), jnp.float32),
        pltpu.VMEM((CHUNK, D_HALF), jnp.float32),
        pltpu.VMEM((CHUNK, D_HALF), jnp.float32),
        pltpu.VMEM((CHUNK, D_HALF), jnp.float32),
        pltpu.VMEM((CHUNK, D_HALF), jnp.float32),
        pltpu.VMEM((CHUNK, D_EDGE), jnp.float32),
        pltpu.VMEM_SHARED((NPAD, D_HALF), jnp.float32),
        pltpu.VMEM_SHARED((NPAD, D_EDGE), jnp.float32),
        pltpu.SemaphoreType.DMA,
        pltpu.SemaphoreType.DMA,
        pltpu.SemaphoreType.DMA,
        pltpu.SemaphoreType.DMA,
        pltpu.SemaphoreType.DMA,
    ],
)(_sc_body)


def kernel(x, edge_index, edge_attr, W_msg, W_self):
    wxT = W_msg[:, :D_IN].T
    weT = W_msg[:, D_IN:].T
    wselfT = W_self.T
    src3d = edge_index[0].reshape(NS, STEPS, CHUNK)
    dst3d = edge_index[1].reshape(NS, STEPS, CHUNK)
    ea3d = edge_attr.reshape(NROWS2D, CHUNK, D_EDGE)
    zp = jnp.zeros((NPAD, D_HALF), jnp.float32)
    zs = jnp.zeros((NPAD, D_EDGE), jnp.float32)

    y0, y1 = _tc_matmul_halves(x, wxT)
    p, s = _sc_scatter(y0, y1, src3d, dst3d, ea3d, zp, zs)
    return _tc_combine(p[0, :N], p[1, :N], s[0, :N], s[1, :N],
                       x, wselfT, weT)


# speedup vs baseline: 5.9968x; 1.1025x over previous
import functools

import jax
import jax.numpy as jnp
from jax import lax
from jax.experimental import pallas as pl
from jax.experimental.pallas import tpu as pltpu
from jax.experimental.pallas import tpu_sc as plsc

N = 10000
E = 320000
D_IN = 128
D_EDGE = 16
D_OUT = 128
D_HALF = D_OUT // 2

NC = 2
NS = 16
E_PER_SUB = E // NS
CHUNK = 80
STEPS = E_PER_SUB // CHUNK
K = 3
NB = 2 * K
NROWS2D = E // CHUNK
ROWS_PER_SUB = N // NS

BLK = 1000
GRID = N // BLK


def _mm_body(x_ref, w0_ref, w1_ref, o0_ref, o1_ref):
    xv = x_ref[...]
    o0_ref[...] = jnp.dot(xv, w0_ref[...], preferred_element_type=jnp.float32)
    o1_ref[...] = jnp.dot(xv, w1_ref[...], preferred_element_type=jnp.float32)


def _tc_matmul_halves(x, wT):
    return pl.pallas_call(
        _mm_body,
        grid=(GRID,),
        in_specs=[
            pl.BlockSpec((BLK, D_IN), lambda i: (i, 0)),
            pl.BlockSpec((D_IN, D_HALF), lambda i: (0, 0)),
            pl.BlockSpec((D_IN, D_HALF), lambda i: (0, 0)),
        ],
        out_specs=[
            pl.BlockSpec((BLK, D_HALF), lambda i: (i, 0)),
            pl.BlockSpec((BLK, D_HALF), lambda i: (i, 0)),
        ],
        out_shape=[
            jax.ShapeDtypeStruct((N, D_HALF), jnp.float32),
            jax.ShapeDtypeStruct((N, D_HALF), jnp.float32),
        ],
    )(x, wT[:, :D_HALF], wT[:, D_HALF:])


def _combine_body(p0, p1, s0, s1, x_ref, wself, we, o_ref):
    agg = jnp.concatenate([p0[...], p1[...]], axis=-1)
    o_ref[...] = (
        agg
        + jnp.dot(x_ref[...], wself[...], preferred_element_type=jnp.float32)
        + jnp.dot(s0[...] + s1[...], we[...],
                  preferred_element_type=jnp.float32)
    )


def _tc_combine(p0, p1, s0, s1, x, wselfT, weT):
    return pl.pallas_call(
        _combine_body,
        grid=(GRID,),
        in_specs=[
            pl.BlockSpec((BLK, D_HALF), lambda i: (i, 0)),
            pl.BlockSpec((BLK, D_HALF), lambda i: (i, 0)),
            pl.BlockSpec((BLK, D_EDGE), lambda i: (i, 0)),
            pl.BlockSpec((BLK, D_EDGE), lambda i: (i, 0)),
            pl.BlockSpec((BLK, D_IN), lambda i: (i, 0)),
            pl.BlockSpec((D_IN, D_OUT), lambda i: (0, 0)),
            pl.BlockSpec((D_EDGE, D_OUT), lambda i: (0, 0)),
        ],
        out_specs=pl.BlockSpec((BLK, D_OUT), lambda i: (i, 0)),
        out_shape=jax.ShapeDtypeStruct((N, D_OUT), jnp.float32),
    )(p0, p1, s0, s1, x, wselfT, weT)


def _sc_body(y0, y1, src3d, dst3d, ea3d, zp, zs, p_out, s_out, *scr):
    src_v, dst_v = scr[0], scr[1]
    rows = list(scr[2:2 + NB])
    ea_v = scr[2 + NB]
    acc, acc_s = scr[3 + NB], scr[4 + NB]
    gsem = list(scr[5 + NB:5 + 2 * NB])
    ssem = list(scr[5 + 2 * NB:5 + 3 * NB])

    cid = lax.axis_index("c")
    sid = lax.axis_index("s")

    r0 = sid * ROWS_PER_SUB
    pltpu.sync_copy(zp.at[pl.ds(r0, ROWS_PER_SUB)],
                    acc.at[pl.ds(r0, ROWS_PER_SUB)])
    pltpu.sync_copy(zs.at[pl.ds(r0, ROWS_PER_SUB)],
                    acc_s.at[pl.ds(r0, ROWS_PER_SUB)])
    plsc.subcore_barrier()

    c0 = sid * STEPS
    pltpu.sync_copy(src3d.at[sid], src_v)
    pltpu.sync_copy(dst3d.at[sid], dst_v)

    def run_half(y_ref, parity):
        def fire_g(t, b):
            pltpu.async_copy(y_ref.at[src_v.at[t]], rows[b], gsem[b])

        def wait_g(b):
            pltpu.make_async_copy(y_ref.at[src_v.at[0]], rows[b],
                                  gsem[b]).wait()

        def fire_s(t, b):
            pltpu.async_copy(rows[b], acc.at[dst_v.at[t]], ssem[b],
                             add=True)

        def wait_s(b):
            pltpu.make_async_copy(rows[b], acc.at[dst_v.at[0]],
                                  ssem[b]).wait()

        def slot_a(t, b, tpar):
            wait_g(b)
            fire_s(t, b)
            if tpar == parity:
                pltpu.sync_copy(ea3d.at[c0 + t], ea_v)
                pltpu.sync_copy(ea_v, acc_s.at[dst_v.at[t]], add=True)

        M = (STEPS - 2 * K) // NB
        tail1 = range(K + M * NB, STEPS - K)

        for b in range(K):
            fire_g(b, b)
        for t in range(K):
            slot_a(t, t, t % 2)
            fire_g(t + K, (t + K) % NB)

        def body(i, carry):
            base = K + i * NB
            for j in range(NB):
                t = base + j
                a = (K + j) % NB
                c = (a + K) % NB
                slot_a(t, a, (K + j) % 2)
                wait_s(c)
                fire_g(t + K, c)
            return carry

        lax.fori_loop(0, M, body, 0)

        for t in tail1:
            a = t % NB
            c = (a + K) % NB
            slot_a(t, a, t % 2)
            wait_s(c)
            fire_g(t + K, c)
        for j in range(K):
            t = STEPS - K + j
            a = t % NB
            slot_a(t, a, t % 2)
            wait_s((a + K) % NB)
        for j in range(K):
            wait_s((STEPS - K + j) % NB)

    @pl.when(cid == 0)
    def _():
        run_half(y0, 0)

    @pl.when(cid == 1)
    def _():
        run_half(y1, 1)

    plsc.subcore_barrier()

    pltpu.sync_copy(acc.at[pl.ds(r0, ROWS_PER_SUB)],
                    p_out.at[cid, pl.ds(r0, ROWS_PER_SUB)])

    pltpu.sync_copy(acc_s.at[pl.ds(r0, ROWS_PER_SUB)],
                    s_out.at[cid, pl.ds(r0, ROWS_PER_SUB)])


_sc_scatter = functools.partial(
    pl.kernel,
    out_type=[
        jax.ShapeDtypeStruct((NC, N, D_HALF), jnp.float32),
        jax.ShapeDtypeStruct((NC, N, D_EDGE), jnp.float32),
    ],
    mesh=plsc.VectorSubcoreMesh(core_axis_name="c", subcore_axis_name="s"),
    compiler_params=pltpu.CompilerParams(use_tc_tiling_on_sc=False),
    scratch_types=(
        [
            pltpu.VMEM((STEPS, CHUNK), jnp.int32),
            pltpu.VMEM((STEPS, CHUNK), jnp.int32),
        ]
        + [pltpu.VMEM((CHUNK, D_HALF), jnp.float32)] * NB
        + [
            pltpu.VMEM((CHUNK, D_EDGE), jnp.float32),
            pltpu.VMEM_SHARED((N, D_HALF), jnp.float32),
            pltpu.VMEM_SHARED((N, D_EDGE), jnp.float32),
        ]
        + [pltpu.SemaphoreType.DMA] * (2 * NB)
    ),
)(_sc_body)


def kernel(x, edge_index, edge_attr, W_msg, W_self):
    wxT = W_msg[:, :D_IN].T
    weT = W_msg[:, D_IN:].T
    wselfT = W_self.T
    src3d = edge_index[0].reshape(NS, STEPS, CHUNK)
    dst3d = edge_index[1].reshape(NS, STEPS, CHUNK)
    ea3d = edge_attr.reshape(NROWS2D, CHUNK, D_EDGE)
    zp = jnp.zeros((N, D_HALF), jnp.float32)
    zs = jnp.zeros((N, D_EDGE), jnp.float32)

    y0, y1 = _tc_matmul_halves(x, wxT)
    p, s = _sc_scatter(y0, y1, src3d, dst3d, ea3d, zp, zs)
    return _tc_combine(p[0], p[1], s[0], s[1], x, wselfT, weT)


# speedup vs baseline: 6.2544x; 1.0430x over previous
import functools

import jax
import jax.numpy as jnp
from jax import lax
from jax.experimental import pallas as pl
from jax.experimental.pallas import tpu as pltpu
from jax.experimental.pallas import tpu_sc as plsc

N = 10000
E = 320000
D_IN = 128
D_EDGE = 16
D_OUT = 128
D_HALF = D_OUT // 2

NC = 2
NS = 16
E_PER_SUB = E // NS
CHUNK = 80
STEPS = E_PER_SUB // CHUNK
K = 3
NB = 2 * K
NROWS2D = E // CHUNK
ROWS_PER_SUB = N // NS

BLK = 1000
GRID = N // BLK


def _mm_body(x_ref, w_ref, o_ref):
    o_ref[...] = jnp.dot(x_ref[...], w_ref[...],
                         preferred_element_type=jnp.float32)


def _tc_matmul(x, wT):
    return pl.pallas_call(
        _mm_body,
        grid=(GRID,),
        in_specs=[
            pl.BlockSpec((BLK, D_IN), lambda i: (i, 0)),
            pl.BlockSpec((D_IN, D_OUT), lambda i: (0, 0)),
        ],
        out_specs=pl.BlockSpec((BLK, D_OUT), lambda i: (i, 0)),
        out_shape=jax.ShapeDtypeStruct((N, D_OUT), jnp.float32),
    )(x, wT)


def _combine_body(p0, p1, s0, s1, x_ref, wself, we, o_ref):
    agg = jnp.concatenate([p0[...], p1[...]], axis=-1)
    o_ref[...] = (
        agg
        + jnp.dot(x_ref[...], wself[...], preferred_element_type=jnp.float32)
        + jnp.dot(s0[...] + s1[...], we[...],
                  preferred_element_type=jnp.float32)
    )


def _tc_combine(p0, p1, s0, s1, x, wselfT, weT):
    return pl.pallas_call(
        _combine_body,
        grid=(GRID,),
        in_specs=[
            pl.BlockSpec((BLK, D_HALF), lambda i: (i, 0)),
            pl.BlockSpec((BLK, D_HALF), lambda i: (i, 0)),
            pl.BlockSpec((BLK, D_EDGE), lambda i: (i, 0)),
            pl.BlockSpec((BLK, D_EDGE), lambda i: (i, 0)),
            pl.BlockSpec((BLK, D_IN), lambda i: (i, 0)),
            pl.BlockSpec((D_IN, D_OUT), lambda i: (0, 0)),
            pl.BlockSpec((D_EDGE, D_OUT), lambda i: (0, 0)),
        ],
        out_specs=pl.BlockSpec((BLK, D_OUT), lambda i: (i, 0)),
        out_shape=jax.ShapeDtypeStruct((N, D_OUT), jnp.float32),
    )(p0, p1, s0, s1, x, wselfT, weT)


def _sc_body(y2, srcA3d, srcB3d, dst3d, ea, zp, zs, p_out, s_out, *scr):
    src_v, dst_v = scr[0], scr[1]
    rows = list(scr[2:2 + NB])
    ea_v = scr[2 + NB]
    acc, acc_s = scr[3 + NB], scr[4 + NB]
    gsem = list(scr[5 + NB:5 + 2 * NB])
    ssem = list(scr[5 + 2 * NB:5 + 3 * NB])

    cid = lax.axis_index("c")
    sid = lax.axis_index("s")

    r0 = sid * ROWS_PER_SUB
    pltpu.sync_copy(zp.at[pl.ds(r0, ROWS_PER_SUB)],
                    acc.at[pl.ds(r0, ROWS_PER_SUB)])
    pltpu.sync_copy(zs.at[pl.ds(r0, ROWS_PER_SUB)],
                    acc_s.at[pl.ds(r0, ROWS_PER_SUB)])
    plsc.subcore_barrier()

    c0 = sid * STEPS
    @pl.when(cid == 0)
    def _():
        pltpu.sync_copy(srcA3d.at[sid], src_v)

    @pl.when(cid == 1)
    def _():
        pltpu.sync_copy(srcB3d.at[sid], src_v)

    pltpu.sync_copy(dst3d.at[sid], dst_v)

    def run_half(parity):
        def fire_g(t, b):
            pltpu.async_copy(y2.at[src_v.at[t]], rows[b], gsem[b])

        def wait_g(b):
            pltpu.make_async_copy(y2.at[src_v.at[0]], rows[b],
                                  gsem[b]).wait()

        def fire_s(t, b):
            pltpu.async_copy(rows[b], acc.at[dst_v.at[t]], ssem[b],
                             add=True)

        def wait_s(b):
            pltpu.make_async_copy(rows[b], acc.at[dst_v.at[0]],
                                  ssem[b]).wait()

        def slot_a(t, b, tpar):
            wait_g(b)
            fire_s(t, b)
            if tpar == parity:
                pltpu.sync_copy(ea.at[pl.ds((c0 + t) * CHUNK, CHUNK)], ea_v)
                pltpu.sync_copy(ea_v, acc_s.at[dst_v.at[t]], add=True)

        M = (STEPS - 2 * K) // NB
        tail1 = range(K + M * NB, STEPS - K)

        for b in range(K):
            fire_g(b, b)
        for t in range(K):
            slot_a(t, t, t % 2)
            fire_g(t + K, (t + K) % NB)

        def body(i, carry):
            base = K + i * NB
            for j in range(NB):
                t = base + j
                a = (K + j) % NB
                c = (a + K) % NB
                slot_a(t, a, (K + j) % 2)
                wait_s(c)
                fire_g(t + K, c)
            return carry

        lax.fori_loop(0, M, body, 0)

        for t in tail1:
            a = t % NB
            c = (a + K) % NB
            slot_a(t, a, t % 2)
            wait_s(c)
            fire_g(t + K, c)
        for j in range(K):
            t = STEPS - K + j
            a = t % NB
            slot_a(t, a, t % 2)
            wait_s((a + K) % NB)
        for j in range(K):
            wait_s((STEPS - K + j) % NB)

    @pl.when(cid == 0)
    def _():
        run_half(0)

    @pl.when(cid == 1)
    def _():
        run_half(1)

    plsc.subcore_barrier()

    pltpu.sync_copy(acc.at[pl.ds(r0, ROWS_PER_SUB)],
                    p_out.at[cid, pl.ds(r0, ROWS_PER_SUB)])

    pltpu.sync_copy(acc_s.at[pl.ds(r0, ROWS_PER_SUB)],
                    s_out.at[cid, pl.ds(r0, ROWS_PER_SUB)])


_sc_scatter = functools.partial(
    pl.kernel,
    out_type=[
        jax.ShapeDtypeStruct((NC, N, D_HALF), jnp.float32),
        jax.ShapeDtypeStruct((NC, N, D_EDGE), jnp.float32),
    ],
    mesh=plsc.VectorSubcoreMesh(core_axis_name="c", subcore_axis_name="s"),
    compiler_params=pltpu.CompilerParams(use_tc_tiling_on_sc=False),
    scratch_types=(
        [
            pltpu.VMEM((STEPS, CHUNK), jnp.int32),
            pltpu.VMEM((STEPS, CHUNK), jnp.int32),
        ]
        + [pltpu.VMEM((CHUNK, D_HALF), jnp.float32)] * NB
        + [
            pltpu.VMEM((CHUNK, D_EDGE), jnp.float32),
            pltpu.VMEM_SHARED((N, D_HALF), jnp.float32),
            pltpu.VMEM_SHARED((N, D_EDGE), jnp.float32),
        ]
        + [pltpu.SemaphoreType.DMA] * (2 * NB)
    ),
)(_sc_body)


def kernel(x, edge_index, edge_attr, W_msg, W_self):
    wxT = W_msg[:, :D_IN].T
    weT = W_msg[:, D_IN:].T
    wselfT = W_self.T
    src2 = edge_index[0] * 2
    srcA3d = src2.reshape(NS, STEPS, CHUNK)
    srcB3d = (src2 + 1).reshape(NS, STEPS, CHUNK)
    dst3d = edge_index[1].reshape(NS, STEPS, CHUNK)
    zp = jnp.zeros((N, D_HALF), jnp.float32)
    zs = jnp.zeros((N, D_EDGE), jnp.float32)

    y = _tc_matmul(x, wxT)
    y2 = y.reshape(2 * N, D_HALF)
    p, s = _sc_scatter(y2, srcA3d, srcB3d, dst3d, edge_attr, zp, zs)
    return _tc_combine(p[0], p[1], s[0], s[1], x, wselfT, weT)
